# baseline (speedup 1.0000x reference)
import functools
import jax
import jax.numpy as jnp
from jax import lax
from jax.experimental import pallas as pl
from jax.experimental.pallas import tpu as pltpu
from jax.experimental.pallas import tpu_sc as plsc

_CH = 32


def _make_sc(batch, seq_len, d_model, dtype):
    info = plsc.get_sparse_core_info()
    nc, ns = info.num_cores, info.num_subcores
    nw = nc * ns
    rows_per_w = seq_len // nw
    mesh = plsc.VectorSubcoreMesh(core_axis_name="c", subcore_axis_name="s")

    @functools.partial(
        pl.kernel,
        mesh=mesh,
        out_type=jax.ShapeDtypeStruct((batch * seq_len, d_model), dtype),
        scratch_types=[
            pltpu.VMEM((_CH, d_model), dtype),
            pltpu.SemaphoreType.DMA,
        ],
    )
    def k(w_hbm, out_hbm, buf, sem):
        wid = lax.axis_index("s") * nc + lax.axis_index("c")
        base = wid * rows_per_w
        for j in range(rows_per_w // _CH):
            r = base + j * _CH
            pltpu.sync_copy(w_hbm.at[pl.ds(r, _CH), :], buf)
            copies = [
                pltpu.make_async_copy(
                    buf, out_hbm.at[pl.ds(b * seq_len + r, _CH), :], sem
                )
                for b in range(batch)
            ]
            for c in copies:
                c.start()
            for c in copies:
                c.wait()

    return k


def kernel(tokens, W_pos):
    batch, seq_len = tokens.shape
    d_model = W_pos.shape[1]
    flat = _make_sc(batch, seq_len, d_model, W_pos.dtype)(W_pos)
    return flat.reshape(batch, seq_len, d_model)


# speedup vs baseline: 1.4373x; 1.4373x over previous
import jax
import jax.numpy as jnp
from jax.experimental import pallas as pl
from jax.experimental.pallas import tpu as pltpu

_BS = 512


def _bcast_kernel(w_ref, o_ref):
    o_ref[...] = jnp.broadcast_to(w_ref[...][None], o_ref.shape)


def kernel(tokens, W_pos):
    batch, seq_len = tokens.shape
    d_model = W_pos.shape[1]
    grid = seq_len // _BS
    return pl.pallas_call(
        _bcast_kernel,
        grid=(grid,),
        in_specs=[pl.BlockSpec((_BS, d_model), lambda s: (s, 0))],
        out_specs=pl.BlockSpec((batch, _BS, d_model), lambda s: (0, s, 0)),
        out_shape=jax.ShapeDtypeStruct((batch, seq_len, d_model), W_pos.dtype),
        compiler_params=pltpu.CompilerParams(
            dimension_semantics=("parallel",),
        ),
    )(W_pos)
